# baseline (speedup 1.0000x reference)
import functools

import jax
import jax.numpy as jnp
from jax import lax
from jax.experimental import pallas as pl
from jax.experimental.pallas import tpu as pltpu
from jax.experimental.pallas import tpu_sc as plsc

N = 50000
D = 128
E = 600000
Q = 4
QD = 32
SUBC = 16
K = 128
CH = 300
EPT = K * CH
EPAD = EPT * SUBC
ZR = 80
NZCH = N // ZR
MBLK = 2000

import numpy as _np
PERM = _np.arange(2 * 16).reshape(2, 16).T.reshape(-1)


def _mm_quarters(h, W):
    Wq = W.reshape(D, Q, QD).transpose(1, 0, 2)[:, :, PERM]

    def body(h_ref, w_ref, o_ref):
        o_ref[...] = jnp.dot(h_ref[...], w_ref[0],
                             preferred_element_type=jnp.float32
                             ).astype(jnp.bfloat16)
    return pl.pallas_call(
        body,
        grid=(N // MBLK, Q),
        in_specs=[
            pl.BlockSpec((MBLK, D), lambda i, q: (i, 0)),
            pl.BlockSpec((1, D, QD), lambda i, q: (q, 0, 0)),
        ],
        out_specs=pl.BlockSpec((MBLK, QD), lambda i, q: (q * (N // MBLK) + i, 0)),
        out_shape=jax.ShapeDtypeStruct((Q * N, QD), jnp.bfloat16),
    )(h, Wq)


def _mm_quarters_relu(s, W):
    Wqq = W.reshape(Q, QD, Q, QD).transpose(0, 2, 1, 3)[:, :, :, PERM]

    def body(s0, s1, s2, s3, w0, w1, w2, w3, o_ref):
        acc = jnp.zeros((MBLK, QD), jnp.float32)
        for sr, wr in ((s0, w0), (s1, w1), (s2, w2), (s3, w3)):
            acc = acc + jnp.dot(jnp.maximum(sr[...], 0.0), wr[0, 0],
                                preferred_element_type=jnp.float32)
        o_ref[...] = acc.astype(jnp.bfloat16)

    nb = N // MBLK
    s_specs = [pl.BlockSpec((MBLK, QD), functools.partial(
        lambda qi, i, qo: (qi * nb + i, 0), qi)) for qi in range(Q)]
    w_specs = [pl.BlockSpec((1, 1, QD, QD), functools.partial(
        lambda qi, i, qo: (qi, qo, 0, 0), qi)) for qi in range(Q)]
    return pl.pallas_call(
        body,
        grid=(nb, Q),
        in_specs=s_specs + w_specs,
        out_specs=pl.BlockSpec((MBLK, QD), lambda i, qo: (qo * nb + i, 0)),
        out_shape=jax.ShapeDtypeStruct((Q * N, QD), jnp.bfloat16),
    )(s, s, s, s, Wqq, Wqq, Wqq, Wqq)


def _mean_out(h0, s1, s2):
    def body(h0_ref, a0, a1, a2, a3, b0, b1, b2, b3, o_ref):
        r1 = jnp.concatenate([jnp.maximum(a[...], 0.0)
                              for a in (a0, a1, a2, a3)], axis=1)
        r2 = jnp.concatenate([jnp.maximum(b[...], 0.0)
                              for b in (b0, b1, b2, b3)], axis=1)
        o_ref[...] = (h0_ref[...] + r1 + r2) * (1.0 / 3.0)

    nb = N // MBLK
    q_specs = [pl.BlockSpec((MBLK, QD), functools.partial(
        lambda q, i: (q * nb + i, 0), q)) for q in range(Q)]
    return pl.pallas_call(
        body,
        grid=(nb,),
        in_specs=[pl.BlockSpec((MBLK, D), lambda i: (i, 0))] + q_specs + q_specs,
        out_specs=pl.BlockSpec((MBLK, D), lambda i: (i, 0)),
        out_shape=jax.ShapeDtypeStruct((N, D), jnp.float32),
    )(h0, s1, s1, s1, s1, s2, s2, s2, s2)


NSUB = 2
SUP = NSUB * K
CH2 = EPT // SUP
ROWS2 = EPAD // K
RPT2 = EPT // K


def _spmm_sc(row, col4, val, table):
    mesh = plsc.VectorSubcoreMesh(core_axis_name="c", subcore_axis_name="s")

    @functools.partial(
        pl.kernel,
        out_type=jax.ShapeDtypeStruct((Q * N, QD), jnp.float32),
        mesh=mesh,
        scratch_types=[
            pltpu.VMEM((3, NSUB, K), jnp.int32),
            pltpu.VMEM((3, NSUB, K), jnp.int32),
            pltpu.VMEM((3, NSUB, K), jnp.float32),
            pltpu.VMEM((2, NSUB, K, QD), jnp.bfloat16),
            pltpu.VMEM((2, NSUB, K, QD), jnp.float32),
            pltpu.VMEM((ZR, QD), jnp.float32),
            pltpu.VMEM_SHARED((N, QD), jnp.float32),
            pltpu.SemaphoreType.DMA((3,)),
            pltpu.SemaphoreType.DMA((2,)),
            pltpu.SemaphoreType.DMA((2,)),
        ],
        compiler_params=pltpu.CompilerParams(use_tc_tiling_on_sc=False,
                                            needs_layout_passes=False),
    )
    def k(row_hbm, col_hbm, val_hbm, table_hbm, out_hbm,
          colb, rowb, valb, rows, outb, zbuf, acc, sem_l, sem_g, sem_s):
        c = lax.axis_index("c")
        s = lax.axis_index("s")
        tile_r = s * RPT2

        zv = jnp.zeros((16,), jnp.float32)
        def zb_body(i, carry):
            zbuf[i, pl.ds(0, 16)] = zv
            zbuf[i, pl.ds(16, 16)] = zv
            return carry
        lax.fori_loop(0, ZR, zb_body, 0)

        nch = (NZCH - s + SUBC - 1) // SUBC

        def fire_linear(q, sc, b):
            r0 = tile_r + sc * NSUB
            pltpu.async_copy(col_hbm.at[q, pl.ds(r0, NSUB)], colb.at[b],
                             sem_l.at[b])
            pltpu.async_copy(row_hbm.at[pl.ds(r0, NSUB)], rowb.at[b], sem_l.at[b])
            pltpu.async_copy(val_hbm.at[pl.ds(r0, NSUB)], valb.at[b], sem_l.at[b])

        def drain_linear(b):
            for ref in (colb, rowb, valb):
                pltpu.make_async_copy(row_hbm.at[pl.ds(0, NSUB)],
                                      ref.at[b], sem_l.at[b]).wait()

        def drain_scatter(b):
            for j in range(NSUB):
                pltpu.make_async_copy(out_hbm.at[pl.ds(0, K)],
                                      outb.at[b, j], sem_s.at[b]).wait()

        def quarter_body(qq, carry):
            q = c * 2 + qq
            qoff = q * N
            plsc.subcore_barrier()

            def zero_body(i, carry2):
                r0 = pl.multiple_of((s + i * SUBC) * ZR, ZR)
                pltpu.async_copy(zbuf, acc.at[pl.ds(r0, ZR)], sem_l.at[0])
                return carry2
            lax.fori_loop(0, nch, zero_body, 0)
            def zero_drain(i, carry2):
                r0 = pl.multiple_of((s + i * SUBC) * ZR, ZR)
                pltpu.make_async_copy(zbuf, acc.at[pl.ds(r0, ZR)],
                                      sem_l.at[0]).wait()
                return carry2
            lax.fori_loop(0, nch, zero_drain, 0)
            plsc.subcore_barrier()

            fire_linear(q, 0, 0)

            def sc_body(sc, carry2):
                b = jnp.bitwise_and(sc, 1)
                bp = jnp.bitwise_and(sc - 1, 1)
                e3 = lax.rem(sc, 3)
                ep3 = lax.rem(sc + 2, 3)

                @pl.when(sc >= 2)
                def _():
                    drain_scatter(b)

                @pl.when(sc + 1 < CH2)
                def _():
                    fire_linear(q, sc + 1, lax.rem(sc + 1, 3))

                @pl.when(sc < CH2)
                def _():
                    drain_linear(e3)
                    for j in range(NSUB):
                        pltpu.async_copy(table_hbm.at[colb.at[e3, j]],
                                         rows.at[b, j], sem_g.at[b])

                @pl.when(sc >= 1)
                def _():
                    for j in range(NSUB):
                        pltpu.make_async_copy(table_hbm.at[pl.ds(0, K)],
                                              rows.at[bp, j], sem_g.at[bp]).wait()
                    for j in range(NSUB):
                        for g in range(K // 16):
                            v16 = valb[ep3, j, pl.ds(g * 16, 16)]
                            for t in range(16):
                                e = g * 16 + t
                                m = v16[t]
                                x = rows[bp, j, e, pl.ds(0, 32)]
                                av, bv = plsc.unpack(
                                    x, format=plsc.PackFormat.INTERLEAVED)
                                outb[bp, j, e, pl.ds(0, 16)] = av * m
                                outb[bp, j, e, pl.ds(16, 16)] = bv * m
                    for j in range(NSUB):
                        pltpu.async_copy(outb.at[bp, j],
                                         acc.at[rowb.at[ep3, j]],
                                         sem_s.at[bp], add=True)
                return carry2

            lax.fori_loop(0, CH2 + 1, sc_body, 0)
            drain_scatter((CH2 - 1) & 1)
            plsc.subcore_barrier()

            def wb_body(i, carry2):
                r0 = pl.multiple_of((s + i * SUBC) * ZR, ZR)
                o0 = pl.multiple_of(qoff + r0, ZR)
                pltpu.async_copy(acc.at[pl.ds(r0, ZR)],
                                 out_hbm.at[pl.ds(o0, ZR)], sem_l.at[1])
                return carry2
            lax.fori_loop(0, nch, wb_body, 0)
            def wb_drain(i, carry2):
                r0 = pl.multiple_of((s + i * SUBC) * ZR, ZR)
                o0 = pl.multiple_of(qoff + r0, ZR)
                pltpu.make_async_copy(acc.at[pl.ds(r0, ZR)],
                                      out_hbm.at[pl.ds(o0, ZR)],
                                      sem_l.at[1]).wait()
                return carry2
            lax.fori_loop(0, nch, wb_drain, 0)
            return carry

        lax.fori_loop(0, 2, quarter_body, 0)

    return k(row, col4, val, table)


def _pad_edges(edge_index, edge_val):
    pad = EPAD - E
    spread = (jnp.arange(pad, dtype=jnp.int32) * 997) % N
    row = jnp.concatenate([edge_index[0].astype(jnp.int32), spread])
    col = jnp.concatenate([edge_index[1].astype(jnp.int32), spread])
    val = jnp.concatenate([edge_val, jnp.zeros((pad,), jnp.float32)])
    col4 = (col.reshape(1, ROWS2, K)
            + (jnp.arange(Q, dtype=jnp.int32) * N).reshape(Q, 1, 1))
    return row.reshape(ROWS2, K), col4, val.reshape(ROWS2, K)


def _propagate(edge_index, edge_val, h0, W):
    row, col4, val = _pad_edges(edge_index, edge_val)
    y1 = _mm_quarters(h0, W)
    s1 = _spmm_sc(row, col4, val, y1)
    y2 = _mm_quarters_relu(s1, W)
    s2 = _spmm_sc(row, col4, val, y2)
    return _mean_out(h0, s1, s2)


def kernel(u_edge_index, u_edge_val, i_edge_index, i_edge_val,
           user_emb, item_emb, W_u, W_i):
    u = _propagate(u_edge_index, u_edge_val, user_emb.astype(jnp.float32), W_u)
    i = _propagate(i_edge_index, i_edge_val, item_emb.astype(jnp.float32), W_i)
    return (u, i)


# speedup vs baseline: 2.8814x; 2.8814x over previous
import functools

import jax
import jax.numpy as jnp
from jax import lax
from jax.experimental import pallas as pl
from jax.experimental.pallas import tpu as pltpu
from jax.experimental.pallas import tpu_sc as plsc

N = 50000
D = 128
E = 600000
Q = 4
QD = 32
SUBC = 16
K = 128
CH = 300
EPT = K * CH
EPAD = EPT * SUBC
ZR = 80
NZCH = N // ZR
MBLK = 2000


def _mm_quarters(h, W):
    Wq = W.reshape(D, Q, QD).transpose(1, 0, 2)

    def body(h_ref, w_ref, o_ref):
        o_ref[...] = jnp.dot(h_ref[...], w_ref[0],
                             preferred_element_type=jnp.float32)
    return pl.pallas_call(
        body,
        grid=(N // MBLK, Q),
        in_specs=[
            pl.BlockSpec((MBLK, D), lambda i, q: (i, 0)),
            pl.BlockSpec((1, D, QD), lambda i, q: (q, 0, 0)),
        ],
        out_specs=pl.BlockSpec((MBLK, QD), lambda i, q: (q * (N // MBLK) + i, 0)),
        out_shape=jax.ShapeDtypeStruct((Q * N, QD), jnp.float32),
    )(h, Wq)


def _mm_quarters_relu(s, W):
    Wqq = W.reshape(Q, QD, Q, QD).transpose(0, 2, 1, 3)

    def body(s0, s1, s2, s3, w0, w1, w2, w3, o_ref):
        acc = jnp.zeros((MBLK, QD), jnp.float32)
        for sr, wr in ((s0, w0), (s1, w1), (s2, w2), (s3, w3)):
            acc = acc + jnp.dot(jnp.maximum(sr[...], 0.0), wr[0, 0],
                                preferred_element_type=jnp.float32)
        o_ref[...] = acc

    nb = N // MBLK
    s_specs = [pl.BlockSpec((MBLK, QD), functools.partial(
        lambda qi, i, qo: (qi * nb + i, 0), qi)) for qi in range(Q)]
    w_specs = [pl.BlockSpec((1, 1, QD, QD), functools.partial(
        lambda qi, i, qo: (qi, qo, 0, 0), qi)) for qi in range(Q)]
    return pl.pallas_call(
        body,
        grid=(nb, Q),
        in_specs=s_specs + w_specs,
        out_specs=pl.BlockSpec((MBLK, QD), lambda i, qo: (qo * nb + i, 0)),
        out_shape=jax.ShapeDtypeStruct((Q * N, QD), jnp.float32),
    )(s, s, s, s, Wqq, Wqq, Wqq, Wqq)


def _mean_out(h0, s1, s2):
    def body(h0_ref, a0, a1, a2, a3, b0, b1, b2, b3, o_ref):
        r1 = jnp.concatenate([jnp.maximum(a[...], 0.0)
                              for a in (a0, a1, a2, a3)], axis=1)
        r2 = jnp.concatenate([jnp.maximum(b[...], 0.0)
                              for b in (b0, b1, b2, b3)], axis=1)
        o_ref[...] = (h0_ref[...] + r1 + r2) * (1.0 / 3.0)

    nb = N // MBLK
    q_specs = [pl.BlockSpec((MBLK, QD), functools.partial(
        lambda q, i: (q * nb + i, 0), q)) for q in range(Q)]
    return pl.pallas_call(
        body,
        grid=(nb,),
        in_specs=[pl.BlockSpec((MBLK, D), lambda i: (i, 0))] + q_specs + q_specs,
        out_specs=pl.BlockSpec((MBLK, D), lambda i: (i, 0)),
        out_shape=jax.ShapeDtypeStruct((N, D), jnp.float32),
    )(h0, s1, s1, s1, s1, s2, s2, s2, s2)


NSUB = 3
SUP = NSUB * K
CH2 = EPT // SUP
ROWS2 = EPAD // K
RPT2 = EPT // K


def _spmm_sc(row, col4, val, table):
    mesh = plsc.VectorSubcoreMesh(core_axis_name="c", subcore_axis_name="s")

    @functools.partial(
        pl.kernel,
        out_type=jax.ShapeDtypeStruct((Q * N, QD), jnp.float32),
        mesh=mesh,
        scratch_types=[
            pltpu.VMEM((3, NSUB, K), jnp.int32),
            pltpu.VMEM((3, NSUB, K), jnp.int32),
            pltpu.VMEM((3, NSUB, K), jnp.float32),
            pltpu.VMEM((2, NSUB * K, QD), jnp.float32),
            pltpu.VMEM((ZR, QD), jnp.float32),
            pltpu.VMEM_SHARED((N, QD), jnp.float32),
            pltpu.SemaphoreType.DMA((3,)),
            pltpu.SemaphoreType.DMA((2,)),
            pltpu.SemaphoreType.DMA((2,)),
        ],
        compiler_params=pltpu.CompilerParams(use_tc_tiling_on_sc=False),
    )
    def k(row_hbm, col_hbm, val_hbm, table_hbm, out_hbm,
          colb, rowb, valb, rows, zbuf, acc, sem_l, sem_g, sem_s):
        c = lax.axis_index("c")
        s = lax.axis_index("s")
        tile_r = s * RPT2

        zv = jnp.zeros((16,), jnp.float32)
        def zb_body(i, carry):
            zbuf[i, pl.ds(0, 16)] = zv
            zbuf[i, pl.ds(16, 16)] = zv
            return carry
        lax.fori_loop(0, ZR, zb_body, 0)

        nch = (NZCH - s + SUBC - 1) // SUBC

        def fire_linear(q, sc, b):
            r0 = tile_r + sc * NSUB
            pltpu.async_copy(col_hbm.at[q, pl.ds(r0, NSUB)], colb.at[b],
                             sem_l.at[b])
            pltpu.async_copy(row_hbm.at[pl.ds(r0, NSUB)], rowb.at[b], sem_l.at[b])
            pltpu.async_copy(val_hbm.at[pl.ds(r0, NSUB)], valb.at[b], sem_l.at[b])

        def drain_linear(b):
            for ref in (colb, rowb, valb):
                pltpu.make_async_copy(row_hbm.at[pl.ds(0, NSUB)],
                                      ref.at[b], sem_l.at[b]).wait()

        def drain_scatter(b):
            pltpu.make_async_copy(table_hbm.at[pl.ds(0, NSUB * K)],
                                  rows.at[b], sem_s.at[b]).wait()

        def quarter_body(qq, carry):
            q = c * 2 + qq
            qoff = q * N
            plsc.subcore_barrier()

            def zero_body(i, carry2):
                r0 = pl.multiple_of((s + i * SUBC) * ZR, ZR)
                pltpu.async_copy(zbuf, acc.at[pl.ds(r0, ZR)], sem_l.at[0])
                return carry2
            lax.fori_loop(0, nch, zero_body, 0)
            def zero_drain(i, carry2):
                r0 = pl.multiple_of((s + i * SUBC) * ZR, ZR)
                pltpu.make_async_copy(zbuf, acc.at[pl.ds(r0, ZR)],
                                      sem_l.at[0]).wait()
                return carry2
            lax.fori_loop(0, nch, zero_drain, 0)
            plsc.subcore_barrier()

            fire_linear(q, 0, 0)

            def sc_body(sc, carry2):
                b = jnp.bitwise_and(sc, 1)
                bp = jnp.bitwise_and(sc - 1, 1)
                e3 = lax.rem(sc, 3)
                ep3 = lax.rem(sc + 2, 3)

                @pl.when(sc >= 2)
                def _():
                    drain_scatter(b)

                @pl.when(sc + 1 < CH2)
                def _():
                    fire_linear(q, sc + 1, lax.rem(sc + 1, 3))

                @pl.when(sc < CH2)
                def _():
                    drain_linear(e3)
                    for j in range(NSUB):
                        pltpu.async_copy(table_hbm.at[colb.at[e3, j]],
                                         rows.at[b, pl.ds(j * K, K)],
                                         sem_g.at[b])

                @pl.when(sc >= 1)
                def _():
                    pltpu.make_async_copy(table_hbm.at[pl.ds(0, NSUB * K)],
                                          rows.at[bp], sem_g.at[bp]).wait()
                    for j in range(NSUB):
                        for g in range(K // 16):
                            v16 = valb[ep3, j, pl.ds(g * 16, 16)]
                            for t in range(16):
                                e = j * K + g * 16 + t
                                m = v16[t]
                                rows[bp, e, pl.ds(0, 16)] = (
                                    rows[bp, e, pl.ds(0, 16)] * m)
                                rows[bp, e, pl.ds(16, 16)] = (
                                    rows[bp, e, pl.ds(16, 16)] * m)
                    for j in range(NSUB):
                        pltpu.async_copy(rows.at[bp, pl.ds(j * K, K)],
                                         acc.at[rowb.at[ep3, j]],
                                         sem_s.at[bp], add=True)
                return carry2

            lax.fori_loop(0, CH2 + 1, sc_body, 0)
            drain_scatter((CH2 - 1) & 1)
            plsc.subcore_barrier()

            def wb_body(i, carry2):
                r0 = pl.multiple_of((s + i * SUBC) * ZR, ZR)
                o0 = pl.multiple_of(qoff + r0, ZR)
                pltpu.async_copy(acc.at[pl.ds(r0, ZR)],
                                 out_hbm.at[pl.ds(o0, ZR)], sem_l.at[1])
                return carry2
            lax.fori_loop(0, nch, wb_body, 0)
            def wb_drain(i, carry2):
                r0 = pl.multiple_of((s + i * SUBC) * ZR, ZR)
                o0 = pl.multiple_of(qoff + r0, ZR)
                pltpu.make_async_copy(acc.at[pl.ds(r0, ZR)],
                                      out_hbm.at[pl.ds(o0, ZR)],
                                      sem_l.at[1]).wait()
                return carry2
            lax.fori_loop(0, nch, wb_drain, 0)
            return carry

        lax.fori_loop(0, 2, quarter_body, 0)

    return k(row, col4, val, table)


def _pad_edges(edge_index, edge_val):
    pad = EPAD - E
    spread = (jnp.arange(pad, dtype=jnp.int32) * 997) % N
    row = jnp.concatenate([edge_index[0].astype(jnp.int32), spread])
    col = jnp.concatenate([edge_index[1].astype(jnp.int32), spread])
    val = jnp.concatenate([edge_val, jnp.zeros((pad,), jnp.float32)])
    col4 = (col.reshape(1, ROWS2, K)
            + (jnp.arange(Q, dtype=jnp.int32) * N).reshape(Q, 1, 1))
    return row.reshape(ROWS2, K), col4, val.reshape(ROWS2, K)


def _propagate(edge_index, edge_val, h0, W):
    row, col4, val = _pad_edges(edge_index, edge_val)
    y1 = _mm_quarters(h0, W)
    s1 = _spmm_sc(row, col4, val, y1)
    y2 = _mm_quarters_relu(s1, W)
    s2 = _spmm_sc(row, col4, val, y2)
    return _mean_out(h0, s1, s2)


def kernel(u_edge_index, u_edge_val, i_edge_index, i_edge_val,
           user_emb, item_emb, W_u, W_i):
    u = _propagate(u_edge_index, u_edge_val, user_emb.astype(jnp.float32), W_u)
    i = _propagate(i_edge_index, i_edge_val, item_emb.astype(jnp.float32), W_i)
    return (u, i)
